# baseline (speedup 1.0000x reference)
import functools

import jax
import jax.numpy as jnp
from jax import lax
from jax.experimental import pallas as pl
from jax.experimental.pallas import tpu as pltpu
from jax.experimental.pallas import tpu_sc as plsc

NUM_EMB = 1000000
EMB_DIM = 32
BATCH = 16384

_info = plsc.get_sparse_core_info()
_NC, _NS = _info.num_cores, _info.num_subcores
_NW = _NC * _NS
_BPW = BATCH // _NW
_CHUNK = 128
_NCH = _BPW // _CHUNK

_mesh = plsc.VectorSubcoreMesh(core_axis_name="c", subcore_axis_name="s")


@functools.partial(
    pl.kernel,
    mesh=_mesh,
    compiler_params=pltpu.CompilerParams(use_tc_tiling_on_sc=False),
    out_type=jax.ShapeDtypeStruct((BATCH * 2, EMB_DIM), jnp.float32),
    scratch_types=[
        pltpu.VMEM((_NCH, _CHUNK), jnp.int32),
        pltpu.VMEM((_NCH, _CHUNK), jnp.int32),
        pltpu.VMEM((_NCH, _CHUNK), jnp.int32),
        pltpu.VMEM((_NCH, _CHUNK), jnp.int32),
        pltpu.VMEM((_NCH, _CHUNK, EMB_DIM), jnp.float32),
        pltpu.VMEM((_NCH, _CHUNK, EMB_DIM), jnp.float32),
        pltpu.SemaphoreType.DMA,
        pltpu.SemaphoreType.DMA,
        pltpu.SemaphoreType.DMA,
        pltpu.SemaphoreType.DMA,
        pltpu.SemaphoreType.DMA,
    ],
)
def _gather_concat(subj_ids, obj_ids, subj_tbl, obj_tbl, out,
                   sidx, oidx, sdst, odst, sbuf, obuf,
                   g0, g1, g2, g3, ssem):
    wid = lax.axis_index("s") * _NC + lax.axis_index("c")
    lane = lax.iota(jnp.int32, 16)
    gsems = (g0, g1, g2, g3)

    pltpu.sync_copy(subj_ids.at[pl.ds(wid * _NCH, _NCH)], sidx)
    pltpu.sync_copy(obj_ids.at[pl.ds(wid * _NCH, _NCH)], oidx)

    base2 = (wid * _BPW) * 2
    for g in range(_BPW // 16):
        ch, off = g >> 3, (g & 7) * 16
        dst = base2 + 2 * (g * 16 + lane)
        sdst[ch, pl.ds(off, 16)] = dst
        odst[ch, pl.ds(off, 16)] = dst + 1

    gathers = []
    for j in range(_NCH):
        gathers.append((
            pltpu.async_copy(subj_tbl.at[sidx.at[j]], sbuf.at[j], gsems[j]),
            pltpu.async_copy(obj_tbl.at[oidx.at[j]], obuf.at[j], gsems[j]),
        ))
    scatters = []
    for j in range(_NCH):
        for h in gathers[j]:
            h.wait()
        scatters.append(
            pltpu.async_copy(sbuf.at[j], out.at[sdst.at[j]], ssem))
        scatters.append(
            pltpu.async_copy(obuf.at[j], out.at[odst.at[j]], ssem))
    for h in scatters:
        h.wait()


def kernel(entity_types, subj_table, obj_table):
    subj_ids = entity_types[:, 0].reshape(BATCH // 128, 128)
    obj_ids = entity_types[:, 1].reshape(BATCH // 128, 128)
    out = _gather_concat(subj_ids, obj_ids, subj_table, obj_table)
    return out.reshape(BATCH, 2 * EMB_DIM)
